# baseline (speedup 1.0000x reference)
import functools

import jax
import jax.numpy as jnp
from jax import lax
from jax.experimental import pallas as pl
from jax.experimental.pallas import tpu as pltpu
from jax.experimental.pallas import tpu_sc as plsc

DENORM = (1440, 24, 31, 53, 7, 12)
OFFS = (0, 1440, 1464, 1495, 1548, 1555)
TROWS = 1567
FEATURE_DIM = 3
NODE = 10000
MD = 64
BATCH = 16
ROWS = BATCH * NODE
LANES = 16
CHUNK = 80
NCHUNKS = ROWS // CHUNK
NC, NS = 2, 16
NW = NC * NS
NSLOT = 4
CPW = -(-NCHUNKS // NW)
CPW += (-CPW) % NSLOT
NOUTER = CPW // NSLOT


def _body(vals_hbm, table_hbm, spat_hbm, out_hbm,
          table_v, vals_v, acc_v, *sems):
    sem_i = sems[:NSLOT]
    sem_o = sems[NSLOT:]
    wid = lax.axis_index("s") * NC + lax.axis_index("c")

    pltpu.sync_copy(table_hbm, table_v)

    def chunk_coords(j):
        t = j * NW + wid
        tc = jnp.minimum(t, NCHUNKS - 1)
        row0 = tc * CHUNK
        n0 = lax.rem(row0, NODE)
        return t, row0, n0

    def fire_in(s, j):
        _, row0, n0 = chunk_coords(j)
        pltpu.async_copy(vals_hbm.at[:, pl.ds(row0, CHUNK)], vals_v.at[s],
                         sem_i[s])
        pltpu.async_copy(spat_hbm.at[pl.ds(n0, CHUNK)], acc_v.at[s], sem_i[s])

    def wait_in(s):
        pltpu.make_async_copy(vals_hbm.at[:, pl.ds(0, CHUNK)], vals_v.at[s],
                              sem_i[s]).wait()
        pltpu.make_async_copy(spat_hbm.at[pl.ds(0, CHUNK)], acc_v.at[s],
                              sem_i[s]).wait()

    def wait_out(s):
        pltpu.make_async_copy(acc_v.at[s], out_hbm.at[pl.ds(0, CHUNK)],
                              sem_o[s]).wait()

    def compute(s, j):
        t, row0, _ = chunk_coords(j)

        @plsc.parallel_loop(0, CHUNK // LANES, 1, unroll=2)
        def _(g):
            sg = pl.ds(g * LANES, LANES)
            ivecs = [(vals_v[s, i, sg] * DENORM[i]).astype(jnp.int32) + OFFS[i]
                     for i in range(6)]
            for rl in range(LANES):
                i = g * LANES + rl
                r = [ivecs[jt][rl] for jt in range(6)]
                acc = [acc_v[s, i, pl.ds(c4 * LANES, LANES)]
                       for c4 in range(MD // LANES)]
                for jt in range(6):
                    for h in range(MD // (2 * LANES)):
                        packed = table_v[r[jt], pl.ds(h * 2 * LANES, 2 * LANES)]
                        lo, hi = plsc.unpack(packed,
                                             format=plsc.PackFormat.INTERLEAVED)
                        acc[2 * h] = acc[2 * h] + lo
                        acc[2 * h + 1] = acc[2 * h + 1] + hi
                for c4 in range(MD // LANES):
                    acc_v[s, i, pl.ds(c4 * LANES, LANES)] = acc[c4]

        @pl.when(t < NCHUNKS)
        def _():
            pltpu.async_copy(acc_v.at[s], out_hbm.at[pl.ds(row0, CHUNK)],
                             sem_o[s])

    for s in range(NSLOT):
        fire_in(s, s)

    def outer(k, c):
        for s in range(NSLOT):
            j = k * NSLOT + s
            t = j * NW + wid
            wait_in(s)
            compute(s, j)

            @pl.when(k < NOUTER - 1)
            def _():
                @pl.when(t < NCHUNKS)
                def _():
                    wait_out(s)

                fire_in(s, j + NSLOT)
        return c

    lax.fori_loop(0, NOUTER, outer, None)
    for s in range(NSLOT):
        t = ((NOUTER - 1) * NSLOT + s) * NW + wid

        @pl.when(t < NCHUNKS)
        def _():
            wait_out(s)


@jax.jit
def kernel(temporal, spatial_prompt, emb0, emb1, emb2, emb3, emb4, emb5):
    vals = temporal[:, -1, :, FEATURE_DIM:FEATURE_DIM + 6]
    vals_t = vals.reshape(ROWS, 6).T
    table = jnp.concatenate([emb0, emb1, emb2, emb3, emb4, emb5], axis=0)
    tb = table.astype(jnp.bfloat16).reshape(TROWS, 2, 2, LANES)
    tb = tb.transpose(0, 1, 3, 2).reshape(TROWS, MD)

    mesh = plsc.VectorSubcoreMesh(core_axis_name="c", subcore_axis_name="s",
                                  num_cores=NC, num_subcores=NS)
    scratch = (
        pltpu.VMEM((TROWS, MD), jnp.bfloat16),
        pltpu.VMEM((NSLOT, 6, CHUNK), jnp.float32),
        pltpu.VMEM((NSLOT, CHUNK, MD), jnp.float32),
    ) + tuple(pltpu.SemaphoreType.DMA for _ in range(2 * NSLOT))
    out = pl.kernel(
        _body,
        out_type=jax.ShapeDtypeStruct((ROWS, MD), jnp.float32),
        mesh=mesh,
        scratch_types=scratch,
        compiler_params=pltpu.CompilerParams(use_tc_tiling_on_sc=False,
                                             needs_layout_passes=False),
    )(vals_t, tb, spatial_prompt)
    return out.reshape(BATCH, NODE, MD)


# speedup vs baseline: 1.3499x; 1.3499x over previous
import functools

import jax
import jax.numpy as jnp
from jax import lax
from jax.experimental import pallas as pl
from jax.experimental.pallas import tpu as pltpu
from jax.experimental.pallas import tpu_sc as plsc

DENORM = (1440, 24, 31, 53, 7, 12)
OFFS = (0, 1440, 1464, 1495, 1548, 1555)
TROWS = 1567
FEATURE_DIM = 3
NODE = 10000
MD = 64
BATCH = 16
ROWS = BATCH * NODE
LANES = 16
CHUNK = 80
NCHUNKS = ROWS // CHUNK
NC, NS = 2, 16
NW = NC * NS
NSLOT = 2
CPW = -(-NCHUNKS // NW)
CPW += CPW % NSLOT
NOUTER = CPW // NSLOT


def _body(vals_hbm, table_hbm, spat_hbm, out_hbm,
          table_v, vals_v, acc_v, *sems):
    sem_i = sems[:NSLOT]
    sem_o = sems[NSLOT:]
    wid = lax.axis_index("s") * NC + lax.axis_index("c")

    pltpu.sync_copy(table_hbm, table_v)

    def chunk_coords(j):
        t = j * NW + wid
        tc = jnp.minimum(t, NCHUNKS - 1)
        row0 = tc * CHUNK
        n0 = lax.rem(row0, NODE)
        return t, row0, n0

    def fire_in(s, j):
        _, row0, n0 = chunk_coords(j)
        pltpu.async_copy(vals_hbm.at[:, pl.ds(row0, CHUNK)], vals_v.at[s],
                         sem_i[s])
        pltpu.async_copy(spat_hbm.at[pl.ds(n0, CHUNK)], acc_v.at[s], sem_i[s])

    def wait_in(s):
        pltpu.make_async_copy(vals_hbm.at[:, pl.ds(0, CHUNK)], vals_v.at[s],
                              sem_i[s]).wait()
        pltpu.make_async_copy(spat_hbm.at[pl.ds(0, CHUNK)], acc_v.at[s],
                              sem_i[s]).wait()

    def wait_out(s):
        pltpu.make_async_copy(acc_v.at[s], out_hbm.at[pl.ds(0, CHUNK)],
                              sem_o[s]).wait()

    def compute(s, j):
        t, row0, _ = chunk_coords(j)

        @plsc.parallel_loop(0, CHUNK // LANES, 1)
        def _(g):
            sg = pl.ds(g * LANES, LANES)
            ivecs = [(vals_v[s, i, sg] * DENORM[i]).astype(jnp.int32) + OFFS[i]
                     for i in range(6)]
            for rl in range(LANES):
                i = g * LANES + rl
                r = [ivecs[jt][rl] for jt in range(6)]
                acc = [acc_v[s, i, pl.ds(c4 * LANES, LANES)]
                       for c4 in range(MD // LANES)]
                for jt in range(6):
                    for h in range(MD // (2 * LANES)):
                        packed = table_v[r[jt], pl.ds(h * 2 * LANES, 2 * LANES)]
                        lo, hi = plsc.unpack(packed,
                                             format=plsc.PackFormat.INTERLEAVED)
                        acc[2 * h] = acc[2 * h] + lo
                        acc[2 * h + 1] = acc[2 * h + 1] + hi
                for c4 in range(MD // LANES):
                    acc_v[s, i, pl.ds(c4 * LANES, LANES)] = acc[c4]

        @pl.when(t < NCHUNKS)
        def _():
            pltpu.async_copy(acc_v.at[s], out_hbm.at[pl.ds(row0, CHUNK)],
                             sem_o[s])

    for s in range(NSLOT):
        fire_in(s, s)

    def outer(k, c):
        for s in range(NSLOT):
            j = k * NSLOT + s
            t = j * NW + wid
            wait_in(s)
            compute(s, j)

            @pl.when(k < NOUTER - 1)
            def _():
                @pl.when(t < NCHUNKS)
                def _():
                    wait_out(s)

                fire_in(s, j + NSLOT)
        return c

    lax.fori_loop(0, NOUTER, outer, None)
    for s in range(NSLOT):
        t = ((NOUTER - 1) * NSLOT + s) * NW + wid

        @pl.when(t < NCHUNKS)
        def _():
            wait_out(s)


@jax.jit
def kernel(temporal, spatial_prompt, emb0, emb1, emb2, emb3, emb4, emb5):
    vals = temporal[:, -1, :, FEATURE_DIM:FEATURE_DIM + 6]
    vals_t = vals.reshape(ROWS, 6).T
    table = jnp.concatenate([emb0, emb1, emb2, emb3, emb4, emb5], axis=0)
    tb = table.astype(jnp.bfloat16).reshape(TROWS, 2, 2, LANES)
    tb = tb.transpose(0, 1, 3, 2).reshape(TROWS, MD)

    mesh = plsc.VectorSubcoreMesh(core_axis_name="c", subcore_axis_name="s",
                                  num_cores=NC, num_subcores=NS)
    scratch = (
        pltpu.VMEM((TROWS, MD), jnp.bfloat16),
        pltpu.VMEM((NSLOT, 6, CHUNK), jnp.float32),
        pltpu.VMEM((NSLOT, CHUNK, MD), jnp.float32),
    ) + tuple(pltpu.SemaphoreType.DMA for _ in range(2 * NSLOT))
    out = pl.kernel(
        _body,
        out_type=jax.ShapeDtypeStruct((ROWS, MD), jnp.float32),
        mesh=mesh,
        scratch_types=scratch,
        compiler_params=pltpu.CompilerParams(use_tc_tiling_on_sc=False,
                                             needs_layout_passes=False),
    )(vals_t, tb, spatial_prompt)
    return out.reshape(BATCH, NODE, MD)
